# baseline (speedup 1.0000x reference)
import functools
import math

import jax
import jax.numpy as jnp
import numpy as np
from jax import lax
from jax.experimental import pallas as pl
from jax.experimental.pallas import tpu as pltpu
from jax.experimental.pallas import tpu_sc as plsc

L = 16
T = 2 ** 19
F = 2
N_MIN = 2 ** 4
BOUND = 2.0
N_MAX = int(2 ** 11 * BOUND)
N_PTS = 524288
ENC_DIM = L * F
HIDDEN = 64
MASK = T - 1

_B = math.exp((math.log(N_MAX) - math.log(N_MIN)) / (L - 1))
RES = [int(math.floor(N_MIN * (_B ** l))) for l in range(L)]
P2 = int(np.uint32(2654435761).astype(np.int32))
P3 = int(np.uint32(805459861).astype(np.int32))

NC = 2
NS = 16
NW = NC * NS
PTS_W = N_PTS // NW
C = 512
G = C // 16
CHUNKS = PTS_W // C


def _sc_encode(pts, ctab):
    mesh = plsc.VectorSubcoreMesh(
        core_axis_name="c", subcore_axis_name="s", num_cores=NC,
        num_subcores=NS)

    @functools.partial(
        pl.kernel,
        out_type=jax.ShapeDtypeStruct((N_PTS, ENC_DIM), jnp.float32),
        mesh=mesh,
        compiler_params=pltpu.CompilerParams(
            needs_layout_passes=False, use_tc_tiling_on_sc=False),
        scratch_types=[
            pltpu.VMEM((C, 4), jnp.float32),
            pltpu.VMEM((C,), jnp.float32),
            pltpu.VMEM((C,), jnp.float32),
            pltpu.VMEM((C,), jnp.float32),
            pltpu.VMEM((C,), jnp.int32),
            pltpu.VMEM((8 * C,), jnp.int32),
            pltpu.VMEM((8 * C,), jnp.int32),
            pltpu.VMEM((8 * C, 4), jnp.float32),
            pltpu.VMEM((8 * C, 4), jnp.float32),
            pltpu.VMEM((C, ENC_DIM), jnp.float32),
            pltpu.SemaphoreType.DMA,
            pltpu.SemaphoreType.DMA,
        ],
    )
    def enc_kernel(pts_hbm, ctab_hbm, out_hbm, pc, xs, ys, zs, csel,
                   idxa, idxb, rowsa, rowsb, enc_c, sema, semb):
        wid = lax.axis_index("s") * NC + lax.axis_index("c")
        iota = lax.iota(jnp.int32, 16)
        zeros16 = jnp.zeros((16,), jnp.float32)
        idxbufs = (idxa, idxb)
        rowbufs = (rowsa, rowsb)
        sems = (sema, semb)

        def compute_idx(l, dst):
            rf = float(RES[l])
            lbase = l * T

            def body(g, carry):
                s = pl.ds(g * 16, 16)
                cx0 = (xs[s] * rf).astype(jnp.int32)
                cy0 = (ys[s] * rf).astype(jnp.int32)
                cz0 = (zs[s] * rf).astype(jnp.int32)
                hy0 = cy0 * P2
                hz0 = cz0 * P3
                hy1 = hy0 + P2
                hz1 = hz0 + P3
                cx1 = cx0 + 1
                t = (hy0 ^ hz0, hy0 ^ hz1, hy1 ^ hz0, hy1 ^ hz1)
                j = 0
                for cxv in (cx0, cx1):
                    for tyz in t:
                        dst[pl.ds(j * C + g * 16, 16)] = (
                            ((cxv ^ tyz) & MASK) + lbase)
                        j += 1
                return carry

            lax.fori_loop(0, G, body, 0)

        def fire(l):
            compute_idx(l, idxbufs[l % 2])
            return pltpu.async_copy(
                ctab_hbm.at[idxbufs[l % 2]], rowbufs[l % 2], sems[l % 2])

        def accumulate(l, rows):
            rf = float(RES[l])

            def body(g, carry):
                s = pl.ds(g * 16, 16)
                px = xs[s] * rf
                py = ys[s] * rf
                pz = zs[s] * rf
                fx = px - px.astype(jnp.int32).astype(jnp.float32)
                fy = py - py.astype(jnp.int32).astype(jnp.float32)
                fz = pz - pz.astype(jnp.int32).astype(jnp.float32)
                gx = 1.0 - fx
                gy = 1.0 - fy
                gz = 1.0 - fz
                cs = csel[s]
                cs1 = cs + 1
                rbase = g * 16 + iota
                wxy = (gx * gy, gx * fy, fx * gy, fx * fy)
                acc0 = zeros16
                acc1 = zeros16
                j = 0
                for i in range(4):
                    for wz in (gz, fz):
                        w = wxy[i] * wz
                        rv = rbase + (j * C)
                        acc0 = acc0 + w * plsc.load_gather(rows, [rv, cs])
                        acc1 = acc1 + w * plsc.load_gather(rows, [rv, cs1])
                        j += 1
                col0 = jnp.full((16,), 2 * l, jnp.int32)
                plsc.store_scatter(enc_c, [rbase, col0], acc0)
                plsc.store_scatter(enc_c, [rbase, col0 + 1], acc1)
                return carry

            lax.fori_loop(0, G, body, 0)

        def chunk_body(ci, carry):
            base = pl.multiple_of(wid * PTS_W + ci * C, C)
            pltpu.sync_copy(pts_hbm.at[pl.ds(base, C)], pc)

            def prep(g, carry2):
                rbase = g * 16 + iota
                zero16 = jnp.full((16,), 0, jnp.int32)
                x = plsc.load_gather(pc, [rbase, zero16])
                y = plsc.load_gather(pc, [rbase, zero16 + 1])
                z = plsc.load_gather(pc, [rbase, zero16 + 2])
                phi = plsc.load_gather(pc, [rbase, zero16 + 3])
                s = pl.ds(g * 16, 16)
                xs[s] = jnp.minimum(jnp.maximum((x + 2.0) * 0.25, 0.0), 1.0)
                ys[s] = jnp.minimum(jnp.maximum((y + 2.0) * 0.25, 0.0), 1.0)
                zs[s] = jnp.minimum(jnp.maximum((z + 2.0) * 0.25, 0.0), 1.0)
                csel[s] = jnp.where(phi >= 0.0, zero16, zero16 + 2)
                return carry2

            lax.fori_loop(0, G, prep, 0)

            descs = {0: fire(0)}
            for l in range(L):
                if l + 1 < L:
                    descs[l + 1] = fire(l + 1)
                descs[l].wait()
                accumulate(l, rowbufs[l % 2])
            pltpu.sync_copy(enc_c, out_hbm.at[pl.ds(base, C)])
            return carry

        lax.fori_loop(0, CHUNKS, chunk_body, 0)

    return enc_kernel(pts, ctab)


_BN = 4096


def _mlp_body(enc_ref, phi_ref, w1p_ref, b1p_ref, w2p_ref, b2p_ref,
              w1m_ref, b1m_ref, w2m_ref, b2m_ref, out_ref):
    e = enc_ref[...]
    hp = jnp.maximum(
        jnp.dot(e, w1p_ref[...], preferred_element_type=jnp.float32)
        + b1p_ref[...], 0.0)
    sp = jnp.dot(hp, w2p_ref[...], preferred_element_type=jnp.float32) \
        + b2p_ref[...]
    hm = jnp.maximum(
        jnp.dot(e, w1m_ref[...], preferred_element_type=jnp.float32)
        + b1m_ref[...], 0.0)
    sm = jnp.dot(hm, w2m_ref[...], preferred_element_type=jnp.float32) \
        + b2m_ref[...]
    out_ref[...] = jnp.where(phi_ref[...] >= 0.0, sp, sm)


def _tc_mlp(enc, phi_r, W1p, b1p, W2p, b2p, W1m, b1m, W2m, b2m):
    grid = N_PTS // _BN
    full = lambda shape: pl.BlockSpec(shape, lambda i: (0, 0))
    return pl.pallas_call(
        _mlp_body,
        grid=(grid,),
        in_specs=[
            pl.BlockSpec((_BN, ENC_DIM), lambda i: (i, 0)),
            pl.BlockSpec((_BN, 1), lambda i: (i, 0)),
            full((ENC_DIM, HIDDEN)), full((1, HIDDEN)),
            full((HIDDEN, 1)), full((1, 1)),
            full((ENC_DIM, HIDDEN)), full((1, HIDDEN)),
            full((HIDDEN, 1)), full((1, 1)),
        ],
        out_specs=pl.BlockSpec((_BN, 1), lambda i: (i, 0)),
        out_shape=jax.ShapeDtypeStruct((N_PTS, 1), jnp.float32),
    )(enc, phi_r, W1p, b1p.reshape(1, HIDDEN), W2p, b2p.reshape(1, 1),
      W1m, b1m.reshape(1, HIDDEN), W2m, b2m.reshape(1, 1))


def kernel(r, phi_r, table_p, W1p, b1p, W2p, b2p, table_m, W1m, b1m, W2m,
           b2m):
    pts = jnp.concatenate([r, phi_r], axis=1)
    ctab = jnp.concatenate([table_p, table_m], axis=-1).reshape(L * T, 4)
    enc = _sc_encode(pts, ctab)
    return _tc_mlp(enc, phi_r, W1p, b1p, W2p, b2p, W1m, b1m, W2m, b2m)
